# baseline (speedup 1.0000x reference)
import functools

import jax
import jax.numpy as jnp
from jax.experimental import pallas as pl
from jax.experimental.pallas import tpu as pltpu

EPS = 1e-5
NEG_SLOPE = 0.2
IMGS_PER_STEP = 4
BN_IMGS_PER_STEP = 8


def _conv_stats_kernel(x_ref, w_ref, b_ref, y_ref, s_ref, patch_ref, *,
                       cin, cout, h, w):
    hw = h * w
    col = jax.lax.broadcasted_iota(jnp.int32, (1, hw), 1) % w

    for i in range(IMGS_PER_STEP):
        x = x_ref[i].astype(jnp.bfloat16)
        xm = {-1: jnp.where(col != w - 1, x, jnp.bfloat16(0)),
              0: x,
              1: jnp.where(col != 0, x, jnp.bfloat16(0))}
        tap = 0
        for oy in (-1, 0, 1):
            for ox in (-1, 0, 1):
                s = oy * w + ox
                xs = xm[ox]
                if s > 0:
                    sh = jnp.concatenate(
                        [xs[:, s:], jnp.zeros((cin, s), jnp.bfloat16)],
                        axis=1)
                elif s < 0:
                    sh = jnp.concatenate(
                        [jnp.zeros((cin, -s), jnp.bfloat16), xs[:, :s]],
                        axis=1)
                else:
                    sh = xs
                patch_ref[pl.ds(tap * cin, cin), :] = sh
                tap += 1

        acc = jnp.dot(w_ref[...], patch_ref[...],
                      preferred_element_type=jnp.float32)
        y = acc + b_ref[...]
        y_ref[i] = y.astype(jnp.bfloat16)

        s_ref[i] = jnp.concatenate(
            [jnp.sum(y, axis=1, keepdims=True),
             jnp.sum(y * y, axis=1, keepdims=True),
             jnp.zeros((cout, 126), jnp.float32)], axis=1)


def _bn_lrelu_kernel(y_ref, sc_ref, sh_ref, o_ref):
    for i in range(BN_IMGS_PER_STEP):
        y = y_ref[i].astype(jnp.float32)
        out = y * sc_ref[...] + sh_ref[...]
        o_ref[i] = jnp.where(out >= 0, out, NEG_SLOPE * out)


@jax.jit
def _forward(x_nchw, w_oihw, bias, gamma, beta):
    N, Cin, H, W = x_nchw.shape
    Cout = w_oihw.shape[0]
    HW = H * W
    nsteps = N // IMGS_PER_STEP

    x3 = x_nchw.reshape(N, Cin, HW)
    w_slab = jnp.transpose(w_oihw, (0, 2, 3, 1)).reshape(Cout, 9 * Cin)
    w_slab = w_slab.astype(jnp.bfloat16)
    b_col = bias.astype(jnp.float32).reshape(Cout, 1)

    conv_kernel = functools.partial(
        _conv_stats_kernel, cin=Cin, cout=Cout, h=H, w=W)

    y3, stats = pl.pallas_call(
        conv_kernel,
        out_shape=(
            jax.ShapeDtypeStruct((N, Cout, HW), jnp.bfloat16),
            jax.ShapeDtypeStruct((N, Cout, 128), jnp.float32),
        ),
        grid=(nsteps,),
        in_specs=[
            pl.BlockSpec((IMGS_PER_STEP, Cin, HW), lambda n: (n, 0, 0)),
            pl.BlockSpec((Cout, 9 * Cin), lambda n: (0, 0)),
            pl.BlockSpec((Cout, 1), lambda n: (0, 0)),
        ],
        out_specs=(
            pl.BlockSpec((IMGS_PER_STEP, Cout, HW), lambda n: (n, 0, 0)),
            pl.BlockSpec((IMGS_PER_STEP, Cout, 128), lambda n: (n, 0, 0)),
        ),
        scratch_shapes=[
            pltpu.VMEM((9 * Cin, HW), jnp.bfloat16),
        ],
        compiler_params=pltpu.CompilerParams(
            dimension_semantics=("arbitrary",),
            allow_input_fusion=[True, False, False]),
    )(x3, w_slab, b_col)

    totals = jnp.sum(stats, axis=0)
    count = jnp.float32(N * HW)
    mean = totals[:, 0] / count
    var = jnp.maximum(totals[:, 1] / count - mean * mean, 0.0)
    inv_std = jax.lax.rsqrt(var + EPS)
    g = gamma.astype(jnp.float32)
    scale = (g * inv_std).reshape(Cout, 1)
    shift = (beta.astype(jnp.float32) - mean * g * inv_std).reshape(Cout, 1)

    out3 = pl.pallas_call(
        _bn_lrelu_kernel,
        out_shape=jax.ShapeDtypeStruct((N, Cout, HW), jnp.float32),
        grid=(N // BN_IMGS_PER_STEP,),
        in_specs=[
            pl.BlockSpec((BN_IMGS_PER_STEP, Cout, HW), lambda n: (n, 0, 0)),
            pl.BlockSpec((Cout, 1), lambda n: (0, 0)),
            pl.BlockSpec((Cout, 1), lambda n: (0, 0)),
        ],
        out_specs=pl.BlockSpec((BN_IMGS_PER_STEP, Cout, HW),
                               lambda n: (n, 0, 0)),
        compiler_params=pltpu.CompilerParams(
            dimension_semantics=("arbitrary",)),
    )(y3, scale, shift)

    return out3.reshape(N, Cout, H, W).astype(x_nchw.dtype)


def kernel(x_nchw, w_oihw, bias, gamma, beta):
    return _forward(x_nchw, w_oihw, bias, gamma, beta)


# speedup vs baseline: 1.0157x; 1.0157x over previous
import functools

import jax
import jax.numpy as jnp
from jax.experimental import pallas as pl
from jax.experimental.pallas import tpu as pltpu

EPS = 1e-5
NEG_SLOPE = 0.2
IMGS_PER_STEP = 2
BN_IMGS_PER_STEP = 4


def _conv_stats_kernel(x_ref, w_ref, b_ref, y_ref, s_ref, patch_ref, *,
                       cin, cout, h, w):
    hw = h * w
    col = jax.lax.broadcasted_iota(jnp.int32, (1, hw), 1) % w

    for i in range(IMGS_PER_STEP):
        x = x_ref[i].astype(jnp.bfloat16)
        xm = {-1: jnp.where(col != w - 1, x, jnp.bfloat16(0)),
              0: x,
              1: jnp.where(col != 0, x, jnp.bfloat16(0))}
        tap = 0
        for oy in (-1, 0, 1):
            for ox in (-1, 0, 1):
                s = oy * w + ox
                xs = xm[ox]
                if s > 0:
                    sh = jnp.concatenate(
                        [xs[:, s:], jnp.zeros((cin, s), jnp.bfloat16)],
                        axis=1)
                elif s < 0:
                    sh = jnp.concatenate(
                        [jnp.zeros((cin, -s), jnp.bfloat16), xs[:, :s]],
                        axis=1)
                else:
                    sh = xs
                patch_ref[pl.ds(tap * cin, cin), :] = sh
                tap += 1

        acc = jnp.dot(w_ref[...], patch_ref[...],
                      preferred_element_type=jnp.float32)
        y = acc + b_ref[...]
        y_ref[i] = y.astype(jnp.bfloat16)

        s_ref[i] = jnp.concatenate(
            [jnp.sum(y, axis=1, keepdims=True),
             jnp.sum(y * y, axis=1, keepdims=True),
             jnp.zeros((cout, 126), jnp.float32)], axis=1)


def _bn_lrelu_kernel(y_ref, sc_ref, sh_ref, o_ref):
    for i in range(BN_IMGS_PER_STEP):
        y = y_ref[i].astype(jnp.float32)
        out = y * sc_ref[...] + sh_ref[...]
        o_ref[i] = jnp.where(out >= 0, out, NEG_SLOPE * out)


@jax.jit
def _forward(x_nchw, w_oihw, bias, gamma, beta):
    N, Cin, H, W = x_nchw.shape
    Cout = w_oihw.shape[0]
    HW = H * W
    nsteps = N // IMGS_PER_STEP

    x3 = x_nchw.reshape(N, Cin, HW)
    w_slab = jnp.transpose(w_oihw, (0, 2, 3, 1)).reshape(Cout, 9 * Cin)
    w_slab = w_slab.astype(jnp.bfloat16)
    b_col = bias.astype(jnp.float32).reshape(Cout, 1)

    conv_kernel = functools.partial(
        _conv_stats_kernel, cin=Cin, cout=Cout, h=H, w=W)

    y3, stats = pl.pallas_call(
        conv_kernel,
        out_shape=(
            jax.ShapeDtypeStruct((N, Cout, HW), jnp.bfloat16),
            jax.ShapeDtypeStruct((N, Cout, 128), jnp.float32),
        ),
        grid=(nsteps,),
        in_specs=[
            pl.BlockSpec((IMGS_PER_STEP, Cin, HW), lambda n: (n, 0, 0)),
            pl.BlockSpec((Cout, 9 * Cin), lambda n: (0, 0)),
            pl.BlockSpec((Cout, 1), lambda n: (0, 0)),
        ],
        out_specs=(
            pl.BlockSpec((IMGS_PER_STEP, Cout, HW), lambda n: (n, 0, 0)),
            pl.BlockSpec((IMGS_PER_STEP, Cout, 128), lambda n: (n, 0, 0)),
        ),
        scratch_shapes=[
            pltpu.VMEM((9 * Cin, HW), jnp.bfloat16),
        ],
        compiler_params=pltpu.CompilerParams(
            dimension_semantics=("arbitrary",),
            allow_input_fusion=[True, False, False]),
    )(x3, w_slab, b_col)

    totals = jnp.sum(stats, axis=0)
    count = jnp.float32(N * HW)
    mean = totals[:, 0] / count
    var = jnp.maximum(totals[:, 1] / count - mean * mean, 0.0)
    inv_std = jax.lax.rsqrt(var + EPS)
    g = gamma.astype(jnp.float32)
    scale = (g * inv_std).reshape(Cout, 1)
    shift = (beta.astype(jnp.float32) - mean * g * inv_std).reshape(Cout, 1)

    out3 = pl.pallas_call(
        _bn_lrelu_kernel,
        out_shape=jax.ShapeDtypeStruct((N, Cout, HW), jnp.float32),
        grid=(N // BN_IMGS_PER_STEP,),
        in_specs=[
            pl.BlockSpec((BN_IMGS_PER_STEP, Cout, HW), lambda n: (n, 0, 0)),
            pl.BlockSpec((Cout, 1), lambda n: (0, 0)),
            pl.BlockSpec((Cout, 1), lambda n: (0, 0)),
        ],
        out_specs=pl.BlockSpec((BN_IMGS_PER_STEP, Cout, HW),
                               lambda n: (n, 0, 0)),
        compiler_params=pltpu.CompilerParams(
            dimension_semantics=("arbitrary",)),
    )(y3, scale, shift)

    return out3.reshape(N, Cout, H, W).astype(x_nchw.dtype)


def kernel(x_nchw, w_oihw, bias, gamma, beta):
    return _forward(x_nchw, w_oihw, bias, gamma, beta)
